# baseline (speedup 1.0000x reference)
import functools

import jax
import jax.numpy as jnp
from jax import lax
from jax.experimental import pallas as pl
from jax.experimental.pallas import tpu as pltpu
from jax.experimental.pallas import tpu_sc as plsc

STACK = 2
N = 32
I = 8192
D = 64
K = 128
NSEG = N * K
NROWS = N * I

NRG = 8
NCG = 4
CPG = D // NCG
RPT = NROWS // NRG
CH = 1024
NCHUNK = RPT // CH
SPT = NSEG // NRG

_mesh = plsc.VectorSubcoreMesh(core_axis_name="c", subcore_axis_name="s")


@functools.partial(
    pl.kernel,
    mesh=_mesh,
    compiler_params=pltpu.CompilerParams(
        use_tc_tiling_on_sc=False, needs_layout_passes=False),
    out_type=jax.ShapeDtypeStruct((STACK, NSEG, D), jnp.float32),
    scratch_types=[
        pltpu.VMEM((NSEG, CPG), jnp.float32),
        pltpu.VMEM((2, CH), jnp.int32),
        pltpu.VMEM((2, CH, CPG), jnp.float32),
        pltpu.VMEM((NSEG,), jnp.int32),
        pltpu.SMEM((16,), jnp.int32),
        pltpu.VMEM_SHARED((16, NSEG // 4, CPG), jnp.float32),
        pltpu.SemaphoreType.DMA,
        pltpu.SemaphoreType.DMA,
        pltpu.SemaphoreType.DMA,
        pltpu.SemaphoreType.DMA,
    ],
)
def _segmax_kernel(fm_hbm, ci_hbm, out_hbm, acc, idxb, rowb, dup, slots,
                   shared, sem_i0, sem_i1, sem_r0, sem_r1):
    core = lax.axis_index("c")
    sid = lax.axis_index("s")
    cg = 2 * core + (sid % 2)
    rg = sid // 2
    row0 = rg * RPT
    c0 = cg * CPG
    seg0 = rg * SPT
    par = sid % 2
    sem_i = (sem_i0, sem_i1)
    sem_r = (sem_r0, sem_r1)
    lane = lax.iota(jnp.int32, 16)

    for s in range(STACK):
        neg_inf = jnp.full((CPG,), -jnp.inf, dtype=jnp.float32)

        def init_body(i, _):
            acc[i, :] = neg_inf
            return 0
        lax.fori_loop(0, NSEG, init_body, 0, unroll=8)

        def make_copies(buf, chunk):
            off = row0 + chunk * CH
            ci_cp = pltpu.make_async_copy(
                ci_hbm.at[s, pl.ds(off, CH)], idxb.at[buf], sem_i[buf])
            row_cp = pltpu.make_async_copy(
                fm_hbm.at[s, pl.ds(off, CH), pl.ds(c0, CPG)],
                rowb.at[buf], sem_r[buf])
            return ci_cp, row_cp

        def start(buf, chunk):
            ci_cp, row_cp = make_copies(buf, chunk)
            ci_cp.start()
            row_cp.start()

        start(0, 0)
        start(1, 1)

        def chunk_pair(p, _):
            for b in range(2):
                chunk = p * 2 + b
                ci_cp, row_cp = make_copies(b, chunk)
                ci_cp.wait()
                row_cp.wait()

                NG = CH // 16
                bfull = jnp.full((16,), b, dtype=jnp.int32)

                def detect(g):
                    segv = idxb[b, pl.ds(g * 16, 16)]
                    plsc.store_scatter(dup, [segv], lane)
                    back = plsc.load_gather(dup, [segv])
                    nd = plsc.all_reduce_population_count(back == lane)
                    return segv, nd[0] == 16

                def row_body(g, carry):
                    segv, all_distinct = carry
                    nxt_carry = detect(jnp.minimum(g + 1, NG - 1))

                    for l in range(16):
                        slots[l] = segv[l]

                    @pl.when(all_distinct)
                    def _():
                        @plsc.parallel_loop(0, 16, 1, unroll=16)
                        def _lanes(l):
                            seg = slots[l]
                            acc[seg, :] = jnp.maximum(
                                acc[seg, :], rowb[b, g * 16 + l, :])

                    @pl.when(jnp.logical_not(all_distinct))
                    def _():
                        for l in range(16):
                            j = g * 16 + l
                            seg = slots[l]
                            acc[seg, :] = jnp.maximum(
                                acc[seg, :], rowb[b, j, :])
                    return nxt_carry
                lax.fori_loop(0, NG, row_body, detect(0), unroll=2)

                @pl.when(chunk + 2 < NCHUNK)
                def _():
                    start(b, chunk + 2)
            return 0

        lax.fori_loop(0, NCHUNK // 2, chunk_pair, 0)

        QSEG = NSEG // 4
        QPT = QSEG // NRG

        for q in range(4):
            pltpu.sync_copy(acc.at[pl.ds(q * QSEG, QSEG)], shared.at[sid])
            plsc.subcore_barrier()

            myseg = rg * QPT
            for r in range(NRG):
                peer = par + 2 * r
                pltpu.sync_copy(shared.at[peer, pl.ds(myseg, QPT)],
                                rowb.at[0, pl.ds(0, QPT)])

                def red_body(g, _):
                    v = rowb[0, g, :]
                    if r == 0:
                        rowb[1, g, :] = v
                    else:
                        rowb[1, g, :] = jnp.maximum(rowb[1, g, :], v)
                    return 0
                lax.fori_loop(0, QPT, red_body, 0, unroll=8)

            pltpu.sync_copy(
                rowb.at[1, pl.ds(0, QPT)],
                out_hbm.at[s, pl.ds(q * QSEG + myseg, QPT), pl.ds(c0, CPG)])
            plsc.subcore_barrier()


def kernel(feature_matrix_batch, cluster_index):
    fm = feature_matrix_batch.reshape(STACK, NROWS, D)
    out = _segmax_kernel(fm, cluster_index)
    return out.reshape(STACK, N, K, D)


# speedup vs baseline: 1.0019x; 1.0019x over previous
import functools

import jax
import jax.numpy as jnp
from jax import lax
from jax.experimental import pallas as pl
from jax.experimental.pallas import tpu as pltpu
from jax.experimental.pallas import tpu_sc as plsc

STACK = 2
N = 32
I = 8192
D = 64
K = 128
NSEG = N * K
NROWS = N * I

NRG = 8
NCG = 4
CPG = D // NCG
RPT = NROWS // NRG
CH = 1024
NCHUNK = RPT // CH
SPT = NSEG // NRG

_mesh = plsc.VectorSubcoreMesh(core_axis_name="c", subcore_axis_name="s")


@functools.partial(
    pl.kernel,
    mesh=_mesh,
    compiler_params=pltpu.CompilerParams(
        use_tc_tiling_on_sc=False, needs_layout_passes=False),
    out_type=jax.ShapeDtypeStruct((STACK, NSEG, D), jnp.float32),
    scratch_types=[
        pltpu.VMEM((NSEG, CPG), jnp.float32),
        pltpu.VMEM((2, CH), jnp.int32),
        pltpu.VMEM((2, CH, CPG), jnp.float32),
        pltpu.VMEM((NSEG,), jnp.int32),
        pltpu.SMEM((16,), jnp.int32),
        pltpu.VMEM_SHARED((16, NSEG // 4, CPG), jnp.float32),
        pltpu.SemaphoreType.DMA,
        pltpu.SemaphoreType.DMA,
        pltpu.SemaphoreType.DMA,
        pltpu.SemaphoreType.DMA,
    ],
)
def _segmax_kernel(fm_hbm, ci_hbm, out_hbm, acc, idxb, rowb, dup, slots,
                   shared, sem_i0, sem_i1, sem_r0, sem_r1):
    core = lax.axis_index("c")
    sid = lax.axis_index("s")
    cg = 2 * core + (sid % 2)
    rg = sid // 2
    row0 = rg * RPT
    c0 = cg * CPG
    seg0 = rg * SPT
    par = sid % 2
    sem_i = (sem_i0, sem_i1)
    sem_r = (sem_r0, sem_r1)
    lane = lax.iota(jnp.int32, 16)

    for s in range(STACK):
        neg_inf = jnp.full((CPG,), -jnp.inf, dtype=jnp.float32)

        @plsc.parallel_loop(0, NSEG, 1, unroll=8)
        def _init(i):
            acc[i, :] = neg_inf

        def make_copies(buf, chunk):
            off = row0 + chunk * CH
            ci_cp = pltpu.make_async_copy(
                ci_hbm.at[s, pl.ds(off, CH)], idxb.at[buf], sem_i[buf])
            row_cp = pltpu.make_async_copy(
                fm_hbm.at[s, pl.ds(off, CH), pl.ds(c0, CPG)],
                rowb.at[buf], sem_r[buf])
            return ci_cp, row_cp

        def start(buf, chunk):
            ci_cp, row_cp = make_copies(buf, chunk)
            ci_cp.start()
            row_cp.start()

        start(0, 0)
        start(1, 1)

        def chunk_pair(p, _):
            for b in range(2):
                chunk = p * 2 + b
                ci_cp, row_cp = make_copies(b, chunk)
                ci_cp.wait()
                row_cp.wait()

                NG = CH // 16
                bfull = jnp.full((16,), b, dtype=jnp.int32)

                def detect(g):
                    segv = idxb[b, pl.ds(g * 16, 16)]
                    plsc.store_scatter(dup, [segv], lane)
                    back = plsc.load_gather(dup, [segv])
                    nd = plsc.all_reduce_population_count(back == lane)
                    return segv, nd[0] == 16

                def row_body(g, carry):
                    segv, all_distinct = carry
                    nxt_carry = detect(jnp.minimum(g + 1, NG - 1))

                    for l in range(16):
                        slots[l] = segv[l]

                    @pl.when(all_distinct)
                    def _():
                        @plsc.parallel_loop(0, 16, 1, unroll=16)
                        def _lanes(l):
                            seg = slots[l]
                            acc[seg, :] = jnp.maximum(
                                acc[seg, :], rowb[b, g * 16 + l, :])

                    @pl.when(jnp.logical_not(all_distinct))
                    def _():
                        for l in range(16):
                            j = g * 16 + l
                            seg = slots[l]
                            acc[seg, :] = jnp.maximum(
                                acc[seg, :], rowb[b, j, :])
                    return nxt_carry
                lax.fori_loop(0, NG, row_body, detect(0))

                @pl.when(chunk + 2 < NCHUNK)
                def _():
                    start(b, chunk + 2)
            return 0

        lax.fori_loop(0, NCHUNK // 2, chunk_pair, 0)

        QSEG = NSEG // 4
        QPT = QSEG // NRG

        for q in range(4):
            pltpu.sync_copy(acc.at[pl.ds(q * QSEG, QSEG)], shared.at[sid])
            plsc.subcore_barrier()

            myseg = rg * QPT
            for r in range(NRG):
                peer = par + 2 * r
                pltpu.sync_copy(shared.at[peer, pl.ds(myseg, QPT)],
                                rowb.at[0, pl.ds(0, QPT)])

                rr = r

                @plsc.parallel_loop(0, QPT, 1, unroll=8)
                def _red(g):
                    v = rowb[0, g, :]
                    if rr == 0:
                        rowb[1, g, :] = v
                    else:
                        rowb[1, g, :] = jnp.maximum(rowb[1, g, :], v)

            pltpu.sync_copy(
                rowb.at[1, pl.ds(0, QPT)],
                out_hbm.at[s, pl.ds(q * QSEG + myseg, QPT), pl.ds(c0, CPG)])
            plsc.subcore_barrier()


def kernel(feature_matrix_batch, cluster_index):
    fm = feature_matrix_batch.reshape(STACK, NROWS, D)
    out = _segmax_kernel(fm, cluster_index)
    return out.reshape(STACK, N, K, D)


# speedup vs baseline: 1.2948x; 1.2924x over previous
import functools

import jax
import jax.numpy as jnp
from jax import lax
from jax.experimental import pallas as pl
from jax.experimental.pallas import tpu as pltpu
from jax.experimental.pallas import tpu_sc as plsc

STACK = 2
N = 32
I = 8192
D = 64
K = 128
NSEG = N * K
NROWS = N * I

NRG = 8
NCG = 4
CPG = D // NCG
RPT = NROWS // NRG
CH = 1024
NCHUNK = RPT // CH
SPT = NSEG // NRG

_mesh = plsc.VectorSubcoreMesh(core_axis_name="c", subcore_axis_name="s")


@functools.partial(
    pl.kernel,
    mesh=_mesh,
    compiler_params=pltpu.CompilerParams(
        use_tc_tiling_on_sc=False, needs_layout_passes=False),
    out_type=jax.ShapeDtypeStruct((STACK, NSEG, D), jnp.float32),
    scratch_types=[
        pltpu.VMEM((NSEG, CPG), jnp.float32),
        pltpu.VMEM((2, CH), jnp.int32),
        pltpu.VMEM((2, CH, CPG), jnp.float32),
        pltpu.VMEM((NSEG,), jnp.int32),
        pltpu.SMEM((16,), jnp.int32),
        pltpu.VMEM_SHARED((16, NSEG // 4, CPG), jnp.float32),
        pltpu.SemaphoreType.DMA,
        pltpu.SemaphoreType.DMA,
        pltpu.SemaphoreType.DMA,
        pltpu.SemaphoreType.DMA,
    ],
)
def _segmax_kernel(fm_hbm, ci_hbm, out_hbm, acc, idxb, rowb, dup, slots,
                   shared, sem_i0, sem_i1, sem_r0, sem_r1):
    core = lax.axis_index("c")
    sid = lax.axis_index("s")
    cg = 2 * core + (sid % 2)
    rg = sid // 2
    row0 = rg * RPT
    c0 = cg * CPG
    seg0 = rg * SPT
    par = sid % 2
    sem_i = (sem_i0, sem_i1)
    sem_r = (sem_r0, sem_r1)
    lane = lax.iota(jnp.int32, 16)

    for s in range(STACK):
        neg_inf = jnp.full((CPG,), -jnp.inf, dtype=jnp.float32)

        @plsc.parallel_loop(0, NSEG, 1, unroll=8)
        def _init(i):
            acc[i, :] = neg_inf

        def make_copies(buf, chunk):
            off = row0 + chunk * CH
            ci_cp = pltpu.make_async_copy(
                ci_hbm.at[s, pl.ds(off, CH)], idxb.at[buf], sem_i[buf])
            row_cp = pltpu.make_async_copy(
                fm_hbm.at[s, pl.ds(off, CH), pl.ds(c0, CPG)],
                rowb.at[buf], sem_r[buf])
            return ci_cp, row_cp

        def start(buf, chunk):
            ci_cp, row_cp = make_copies(buf, chunk)
            ci_cp.start()
            row_cp.start()

        start(0, 0)
        start(1, 1)

        def chunk_pair(p, _):
            for b in range(2):
                chunk = p * 2 + b
                ci_cp, row_cp = make_copies(b, chunk)
                ci_cp.wait()
                row_cp.wait()

                NG = CH // 16

                def row_body(g, _):
                    segv = idxb[b, pl.ds(g * 16, 16)]

                    plsc.store_scatter(dup, [segv], lane)
                    back = plsc.load_gather(dup, [segv])
                    nd = plsc.all_reduce_population_count(back == lane)

                    for l in range(16):
                        slots[l] = segv[l]

                    @plsc.parallel_loop(0, 16, 1, unroll=16)
                    def _lanes(l):
                        seg = slots[l]
                        acc[seg, :] = jnp.maximum(
                            acc[seg, :], rowb[b, g * 16 + l, :])

                    @pl.when(nd[0] != 16)
                    def _():
                        for l in range(16):
                            j = g * 16 + l
                            seg = slots[l]
                            acc[seg, :] = jnp.maximum(
                                acc[seg, :], rowb[b, j, :])
                    return 0
                lax.fori_loop(0, NG, row_body, 0)

                @pl.when(chunk + 2 < NCHUNK)
                def _():
                    start(b, chunk + 2)
            return 0

        lax.fori_loop(0, NCHUNK // 2, chunk_pair, 0)

        QSEG = NSEG // 4
        QPT = QSEG // NRG

        for q in range(4):
            pltpu.sync_copy(acc.at[pl.ds(q * QSEG, QSEG)], shared.at[sid])
            plsc.subcore_barrier()

            myseg = rg * QPT
            for r in range(NRG):
                peer = par + 2 * r
                pltpu.sync_copy(shared.at[peer, pl.ds(myseg, QPT)],
                                rowb.at[0, pl.ds(0, QPT)])

                rr = r

                @plsc.parallel_loop(0, QPT, 1, unroll=8)
                def _red(g):
                    v = rowb[0, g, :]
                    if rr == 0:
                        rowb[1, g, :] = v
                    else:
                        rowb[1, g, :] = jnp.maximum(rowb[1, g, :], v)

            pltpu.sync_copy(
                rowb.at[1, pl.ds(0, QPT)],
                out_hbm.at[s, pl.ds(q * QSEG + myseg, QPT), pl.ds(c0, CPG)])
            plsc.subcore_barrier()


def kernel(feature_matrix_batch, cluster_index):
    fm = feature_matrix_batch.reshape(STACK, NROWS, D)
    out = _segmax_kernel(fm, cluster_index)
    return out.reshape(STACK, N, K, D)


# speedup vs baseline: 1.3208x; 1.0200x over previous
import functools

import jax
import jax.numpy as jnp
from jax import lax
from jax.experimental import pallas as pl
from jax.experimental.pallas import tpu as pltpu
from jax.experimental.pallas import tpu_sc as plsc

STACK = 2
N = 32
I = 8192
D = 64
K = 128
NSEG = N * K
NROWS = N * I

NRG = 8
NCG = 4
CPG = D // NCG
RPT = NROWS // NRG
CH = 1024
NCHUNK = RPT // CH
SPT = NSEG // NRG

_mesh = plsc.VectorSubcoreMesh(core_axis_name="c", subcore_axis_name="s")


@functools.partial(
    pl.kernel,
    mesh=_mesh,
    compiler_params=pltpu.CompilerParams(
        use_tc_tiling_on_sc=False, needs_layout_passes=False),
    out_type=jax.ShapeDtypeStruct((STACK, NSEG, D), jnp.float32),
    scratch_types=[
        pltpu.VMEM((NSEG, CPG), jnp.float32),
        pltpu.VMEM((2, CH), jnp.int32),
        pltpu.VMEM((2, CH, CPG), jnp.float32),
        pltpu.VMEM((NSEG,), jnp.int32),
        pltpu.SMEM((16,), jnp.int32),
        pltpu.VMEM_SHARED((16, NSEG // 4, CPG), jnp.float32),
        pltpu.SemaphoreType.DMA,
        pltpu.SemaphoreType.DMA,
        pltpu.SemaphoreType.DMA,
        pltpu.SemaphoreType.DMA,
    ],
)
def _segmax_kernel(fm_hbm, ci_hbm, out_hbm, acc, idxb, rowb, dup, slots,
                   shared, sem_i0, sem_i1, sem_r0, sem_r1):
    core = lax.axis_index("c")
    sid = lax.axis_index("s")
    cg = 2 * core + (sid % 2)
    rg = sid // 2
    row0 = rg * RPT
    c0 = cg * CPG
    seg0 = rg * SPT
    par = sid % 2
    sem_i = (sem_i0, sem_i1)
    sem_r = (sem_r0, sem_r1)
    lane = lax.iota(jnp.int32, 16)

    for s in range(STACK):
        neg_inf = jnp.full((CPG,), -jnp.inf, dtype=jnp.float32)

        @plsc.parallel_loop(0, NSEG, 1, unroll=8)
        def _init(i):
            acc[i, :] = neg_inf

        def make_copies(buf, chunk):
            off = row0 + chunk * CH
            ci_cp = pltpu.make_async_copy(
                ci_hbm.at[s, pl.ds(off, CH)], idxb.at[buf], sem_i[buf])
            row_cp = pltpu.make_async_copy(
                fm_hbm.at[s, pl.ds(off, CH), pl.ds(c0, CPG)],
                rowb.at[buf], sem_r[buf])
            return ci_cp, row_cp

        def start(buf, chunk):
            ci_cp, row_cp = make_copies(buf, chunk)
            ci_cp.start()
            row_cp.start()

        start(0, 0)
        start(1, 1)

        def chunk_pair(p, _):
            for b in range(2):
                chunk = p * 2 + b
                ci_cp, row_cp = make_copies(b, chunk)
                ci_cp.wait()
                row_cp.wait()

                NG = CH // 16

                def row_body(g, _):
                    segv = idxb[b, pl.ds(g * 16, 16)]

                    plsc.store_scatter(dup, [segv], lane)
                    back = plsc.load_gather(dup, [segv])
                    nd = plsc.all_reduce_population_count(back == lane)

                    for l in range(16):
                        slots[l] = segv[l]

                    @plsc.parallel_loop(0, 16, 1, unroll=16)
                    def _lanes(l):
                        seg = slots[l]
                        acc[seg, :] = jnp.maximum(
                            acc[seg, :], rowb[b, g * 16 + l, :])

                    @pl.when(nd[0] != 16)
                    def _():
                        for l in range(16):
                            j = g * 16 + l
                            seg = slots[l]
                            acc[seg, :] = jnp.maximum(
                                acc[seg, :], rowb[b, j, :])
                    return 0
                lax.fori_loop(0, NG, row_body, 0, unroll=2)

                @pl.when(chunk + 2 < NCHUNK)
                def _():
                    start(b, chunk + 2)
            return 0

        lax.fori_loop(0, NCHUNK // 2, chunk_pair, 0)

        QSEG = NSEG // 4
        QPT = QSEG // NRG

        for q in range(4):
            pltpu.sync_copy(acc.at[pl.ds(q * QSEG, QSEG)], shared.at[sid])
            plsc.subcore_barrier()

            myseg = rg * QPT
            for r in range(NRG):
                peer = par + 2 * r
                pltpu.sync_copy(shared.at[peer, pl.ds(myseg, QPT)],
                                rowb.at[0, pl.ds(0, QPT)])

                rr = r

                @plsc.parallel_loop(0, QPT, 1, unroll=8)
                def _red(g):
                    v = rowb[0, g, :]
                    if rr == 0:
                        rowb[1, g, :] = v
                    else:
                        rowb[1, g, :] = jnp.maximum(rowb[1, g, :], v)

            pltpu.sync_copy(
                rowb.at[1, pl.ds(0, QPT)],
                out_hbm.at[s, pl.ds(q * QSEG + myseg, QPT), pl.ds(c0, CPG)])
            plsc.subcore_barrier()


def kernel(feature_matrix_batch, cluster_index):
    fm = feature_matrix_batch.reshape(STACK, NROWS, D)
    out = _segmax_kernel(fm, cluster_index)
    return out.reshape(STACK, N, K, D)


# speedup vs baseline: 1.3282x; 1.0056x over previous
import functools

import jax
import jax.numpy as jnp
from jax import lax
from jax.experimental import pallas as pl
from jax.experimental.pallas import tpu as pltpu
from jax.experimental.pallas import tpu_sc as plsc

STACK = 2
N = 32
I = 8192
D = 64
K = 128
NSEG = N * K
NROWS = N * I

NRG = 8
NCG = 4
CPG = D // NCG
RPT = NROWS // NRG
CH = 1024
NCHUNK = RPT // CH
SPT = NSEG // NRG

_mesh = plsc.VectorSubcoreMesh(core_axis_name="c", subcore_axis_name="s")


@functools.partial(
    pl.kernel,
    mesh=_mesh,
    compiler_params=pltpu.CompilerParams(
        use_tc_tiling_on_sc=False, needs_layout_passes=False),
    out_type=jax.ShapeDtypeStruct((STACK, NSEG, D), jnp.float32),
    scratch_types=[
        pltpu.VMEM((NSEG, CPG), jnp.float32),
        pltpu.VMEM((2, CH), jnp.int32),
        pltpu.VMEM((2, CH, CPG), jnp.float32),
        pltpu.VMEM((NSEG,), jnp.int32),
        pltpu.SMEM((16,), jnp.int32),
        pltpu.VMEM_SHARED((16, NSEG // 4, CPG), jnp.float32),
        pltpu.SemaphoreType.DMA,
        pltpu.SemaphoreType.DMA,
        pltpu.SemaphoreType.DMA,
        pltpu.SemaphoreType.DMA,
    ],
)
def _segmax_kernel(fm_hbm, ci_hbm, out_hbm, acc, idxb, rowb, dup, slots,
                   shared, sem_i0, sem_i1, sem_r0, sem_r1):
    core = lax.axis_index("c")
    sid = lax.axis_index("s")
    cg = 2 * core + (sid % 2)
    rg = sid // 2
    row0 = rg * RPT
    c0 = cg * CPG
    seg0 = rg * SPT
    par = sid % 2
    sem_i = (sem_i0, sem_i1)
    sem_r = (sem_r0, sem_r1)
    lane = lax.iota(jnp.int32, 16)

    for s in range(STACK):
        neg_inf = jnp.full((CPG,), -jnp.inf, dtype=jnp.float32)

        @plsc.parallel_loop(0, NSEG, 1, unroll=8)
        def _init(i):
            acc[i, :] = neg_inf

        def make_copies(buf, chunk):
            off = row0 + chunk * CH
            ci_cp = pltpu.make_async_copy(
                ci_hbm.at[s, pl.ds(off, CH)], idxb.at[buf], sem_i[buf])
            row_cp = pltpu.make_async_copy(
                fm_hbm.at[s, pl.ds(off, CH), pl.ds(c0, CPG)],
                rowb.at[buf], sem_r[buf])
            return ci_cp, row_cp

        def start(buf, chunk):
            ci_cp, row_cp = make_copies(buf, chunk)
            ci_cp.start()
            row_cp.start()

        start(0, 0)
        start(1, 1)

        def chunk_pair(p, _):
            for b in range(2):
                chunk = p * 2 + b
                ci_cp, row_cp = make_copies(b, chunk)
                ci_cp.wait()
                row_cp.wait()

                NG = CH // 16

                def row_body(g, _):
                    segv = idxb[b, pl.ds(g * 16, 16)]

                    plsc.store_scatter(dup, [segv], lane)
                    back = plsc.load_gather(dup, [segv])
                    nd = plsc.all_reduce_population_count(back == lane)

                    for l in range(16):
                        slots[l] = segv[l]

                    @plsc.parallel_loop(0, 16, 1, unroll=16)
                    def _lanes(l):
                        seg = slots[l]
                        acc[seg, :] = jnp.maximum(
                            acc[seg, :], rowb[b, g * 16 + l, :])

                    @pl.when(nd[0] != 16)
                    def _():
                        for l in range(16):
                            j = g * 16 + l
                            seg = slots[l]
                            acc[seg, :] = jnp.maximum(
                                acc[seg, :], rowb[b, j, :])
                    return 0
                lax.fori_loop(0, NG, row_body, 0, unroll=4)

                @pl.when(chunk + 2 < NCHUNK)
                def _():
                    start(b, chunk + 2)
            return 0

        lax.fori_loop(0, NCHUNK // 2, chunk_pair, 0)

        QSEG = NSEG // 4
        QPT = QSEG // NRG

        for q in range(4):
            pltpu.sync_copy(acc.at[pl.ds(q * QSEG, QSEG)], shared.at[sid])
            plsc.subcore_barrier()

            myseg = rg * QPT
            for r in range(NRG):
                peer = par + 2 * r
                pltpu.sync_copy(shared.at[peer, pl.ds(myseg, QPT)],
                                rowb.at[0, pl.ds(0, QPT)])

                rr = r

                @plsc.parallel_loop(0, QPT, 1, unroll=8)
                def _red(g):
                    v = rowb[0, g, :]
                    if rr == 0:
                        rowb[1, g, :] = v
                    else:
                        rowb[1, g, :] = jnp.maximum(rowb[1, g, :], v)

            pltpu.sync_copy(
                rowb.at[1, pl.ds(0, QPT)],
                out_hbm.at[s, pl.ds(q * QSEG + myseg, QPT), pl.ds(c0, CPG)])
            plsc.subcore_barrier()


def kernel(feature_matrix_batch, cluster_index):
    fm = feature_matrix_batch.reshape(STACK, NROWS, D)
    out = _segmax_kernel(fm, cluster_index)
    return out.reshape(STACK, N, K, D)
